# baseline (speedup 1.0000x reference)
import jax
import jax.numpy as jnp
from jax import lax
from jax.experimental import pallas as pl
from jax.experimental.pallas import tpu as pltpu
from jax.experimental.pallas import tpu_sc as plsc

BATCH = 4096
SEQ = 50
DIM = 64
LANES = 16

NUM_CORES = 2
NUM_SUBCORES = 16
NW = NUM_CORES * NUM_SUBCORES
BLK = BATCH // NW
NB2 = 2


def _sc_body(xt_hbm, len_hbm, tab_hbm, o5_hbm, r4_hbm,
             idx_v, rows_v, outt_v, acc_v, len_v, recip_v, sem_g, sem_o):
    wid = lax.axis_index("s") * NUM_CORES + lax.axis_index("c")
    bc = wid

    pltpu.sync_copy(xt_hbm.at[:, pl.ds(bc * BLK, BLK)], idx_v)
    pltpu.sync_copy(len_hbm.at[pl.ds(bc * BLK, BLK)], len_v)
    for g in range(BLK // LANES):
        sl = pl.ds(g * LANES, LANES)
        recip_v[sl] = 1.0 / len_v[sl].astype(jnp.float32)

    def issue(t, b):
        return pltpu.async_copy(tab_hbm.at[idx_v.at[t]], rows_v.at[b],
                                sem_g[b])

    def transpose_block(t, b, first):
        def dbody(d, carry):
            dr = d // 8
            ds = d % 8
            for g in range(BLK // LANES):
                ridx = g * LANES + lax.iota(jnp.int32, LANES)
                v = plsc.load_gather(
                    rows_v, [jnp.full((LANES,), b, jnp.int32), ridx,
                             jnp.full((LANES,), d, jnp.int32)])
                sl = pl.ds(g * LANES, LANES)
                outt_v[b, dr, ds, sl] = v
                if first:
                    acc_v[dr, ds, sl] = v
                else:
                    acc_v[dr, ds, sl] = acc_v[dr, ds, sl] + v
            return carry

        lax.fori_loop(0, DIM, dbody, 0)

    def store_out(t, b):
        return pltpu.async_copy(outt_v.at[b], o5_hbm.at[t, :, bc], sem_o[b])

    cps_g = {}
    cps_o = {}
    cps_g[0] = issue(0, 0)
    cps_g[1] = issue(1, 1)
    cps_g[0].wait()
    transpose_block(0, 0, first=True)
    cps_o[0] = store_out(0, 0)
    issue(2, 0)
    cps_g[1].wait()
    transpose_block(1, 1, first=False)
    cps_o[1] = store_out(1, 1)
    issue(3, 1)

    def tbody(t2, carry):
        for b in range(NB2):
            t = t2 * NB2 + b
            cps_o[b].wait()
            cps_g[b].wait()
            transpose_block(t, b, first=False)
            store_out(t, b)
            issue(t + NB2, b)
        return carry

    lax.fori_loop(1, SEQ // NB2 - 1, tbody, 0)

    for b in range(NB2):
        t = SEQ - NB2 + b
        cps_o[b].wait()
        cps_g[b].wait()
        transpose_block(t, b, first=False)
        store_out(t, b)
    for b in range(NB2):
        cps_o[b].wait()

    for d in range(DIM):
        dr, ds = divmod(d, 8)
        for g in range(BLK // LANES):
            sl = pl.ds(g * LANES, LANES)
            acc_v[dr, ds, sl] = acc_v[dr, ds, sl] * recip_v[sl]
    pltpu.sync_copy(acc_v, r4_hbm.at[:, bc])


@jax.jit
def _run(xt, x_len, emb_weight):
    mesh = plsc.VectorSubcoreMesh(core_axis_name="c", subcore_axis_name="s")
    k = pl.kernel(
        _sc_body,
        mesh=mesh,
        compiler_params=pltpu.CompilerParams(
            needs_layout_passes=False, use_tc_tiling_on_sc=False),
        out_type=(
            jax.ShapeDtypeStruct((SEQ, 8, NW, 8, BLK), jnp.float32),
            jax.ShapeDtypeStruct((8, NW, 8, BLK), jnp.float32),
        ),
        scratch_types=[
            pltpu.VMEM((SEQ, BLK), jnp.int32),
            pltpu.VMEM((NB2, BLK, DIM), jnp.float32),
            pltpu.VMEM((NB2, 8, 8, BLK), jnp.float32),
            pltpu.VMEM((8, 8, BLK), jnp.float32),
            pltpu.VMEM((BLK,), jnp.int32),
            pltpu.VMEM((BLK,), jnp.float32),
            [pltpu.SemaphoreType.DMA] * NB2,
            [pltpu.SemaphoreType.DMA] * NB2,
        ],
    )
    return k(xt, x_len, emb_weight)


def kernel(x, x_len, emb_weight):
    xt = x.astype(jnp.int32).T
    o5, r4 = _run(xt, x_len.astype(jnp.int32), emb_weight)
    out = o5.transpose((2, 4, 0, 1, 3)).reshape(BATCH, SEQ, DIM)
    ret = r4.transpose((1, 3, 0, 2)).reshape(BATCH, DIM)
    return (ret, out)
